# baseline (speedup 1.0000x reference)
import functools

import jax
import jax.numpy as jnp
from jax import lax
from jax.experimental import pallas as pl
from jax.experimental.pallas import tpu as pltpu
from jax.experimental.pallas import tpu_sc as plsc

N = 10000
E = 320000
D = 128

NC = 2
NS = 16
NW = NC * NS

NPAD = 10240
RT = NPAD // NS
K = 40
EW = E // NW
CW = EW // K
NBUF = 5
GLEAD = 3
DG = 5
DB = CW // DG

RB = 2048
NBLK = NPAD // RB
RBO = RB
NBLKO = NBLK


def _make_agg(with_deg):
    def body(*args):
        if with_deg:
            (x_hbm, edge_hbm,
             p_out, degp_out,
             acc_sh, deg_sh, ibuf, rows_v, ones_v,
             gsem, ssem, dsem, isem, zsem) = args
        else:
            (x_hbm, edge_hbm,
             p_out,
             acc_sh, ibuf, rows_v,
             gsem, ssem, isem, zsem) = args
        cid = lax.axis_index("c")
        sid = lax.axis_index("s")
        wid = sid * NC + cid

        pltpu.sync_copy(edge_hbm.at[0, wid, 0], ibuf.at[0, 0])
        pltpu.sync_copy(edge_hbm.at[1, wid, 0], ibuf.at[0, 1])

        for cn in range(GLEAD):
            pltpu.async_copy(x_hbm.at[ibuf.at[0, 0, cn]], rows_v.at[cn],
                             gsem.at[cn])

        z16 = jnp.zeros((16,), jnp.float32)

        def zfill(i, carry):
            for j in range(D // 16):
                rows_v[NBUF - 1, i, pl.ds(j * 16, 16)] = z16
            return carry

        lax.fori_loop(0, K, zfill, 0)
        if with_deg:
            one16 = jnp.full((16,), 1.0, jnp.float32)
            for off in sorted({min(j * 16, K - 16)
                              for j in range((K + 15) // 16)}):
                ones_v[pl.ds(off, 16)] = one16
        zrow = rows_v.at[NBUF - 1]
        for t in range(RT // K):
            pltpu.async_copy(
                zrow, acc_sh.at[pl.ds(sid * RT + t * K, K)], zsem)
        if with_deg:
            for t in range(RT // D):
                pltpu.async_copy(
                    rows_v.at[NBUF - 1, 0],
                    deg_sh.at[pl.ds(sid * RT + t * D, D)], zsem)
        for t in range(RT // K):
            pltpu.make_async_copy(
                zrow, acc_sh.at[pl.ds(sid * RT, K)], zsem).wait()
        if with_deg:
            for t in range(RT // D):
                pltpu.make_async_copy(
                    rows_v.at[NBUF - 1, 0],
                    deg_sh.at[pl.ds(sid * RT, D)], zsem).wait()
        plsc.subcore_barrier()

        def wait_scatter(slot):
            pltpu.make_async_copy(
                rows_v.at[0], acc_sh.at[ibuf.at[0, 1, 0]],
                ssem.at[slot]).wait()
            if with_deg:
                pltpu.make_async_copy(
                    ones_v, deg_sh.at[ibuf.at[0, 1, 0]],
                    dsem.at[slot]).wait()

        def do_block(g, gb, first):
            for r in range(DG):
                t = r + 3
                st = t % NBUF
                tgb = gb if t < DG else 1 - gb

                if first and r < 2:
                    pass
                else:
                    wait_scatter(st)

                if r == 1:
                    @pl.when(g + 1 < DB)
                    def _():
                        pltpu.async_copy(edge_hbm.at[0, wid, g + 1],
                                         ibuf.at[1 - gb, 0], isem.at[0])
                        pltpu.async_copy(edge_hbm.at[1, wid, g + 1],
                                         ibuf.at[1 - gb, 1], isem.at[1])
                if r == 2:
                    @pl.when(g + 1 < DB)
                    def _():
                        pltpu.make_async_copy(
                            edge_hbm.at[0, wid, 0], ibuf.at[0, 0],
                            isem.at[0]).wait()
                        pltpu.make_async_copy(
                            edge_hbm.at[1, wid, 0], ibuf.at[0, 1],
                            isem.at[1]).wait()

                @pl.when(g * DG + t < CW)
                def _():
                    pltpu.async_copy(
                        x_hbm.at[ibuf.at[tgb, 0, t % DG]],
                        rows_v.at[st], gsem.at[st])

                pltpu.make_async_copy(
                    x_hbm.at[ibuf.at[0, 0, 0]], rows_v.at[r],
                    gsem.at[r]).wait()
                didx = ibuf.at[gb, 1, r]
                pltpu.async_copy(rows_v.at[r], acc_sh.at[didx],
                                 ssem.at[r], add=True)
                if with_deg:
                    pltpu.async_copy(ones_v, deg_sh.at[didx],
                                     dsem.at[r], add=True)

        def pair(t2, carry):
            g = 2 * t2
            do_block(g, 0, False)
            do_block(g + 1, 1, False)
            return carry

        do_block(0, 0, True)
        do_block(1, 1, False)
        lax.fori_loop(1, DB // 2, pair, 0)
        for r in (3, 4):
            wait_scatter(r)
        plsc.subcore_barrier()

        pltpu.sync_copy(acc_sh.at[pl.ds(sid * RT, RT)],
                        p_out.at[cid, pl.ds(sid * RT, RT)])
        if with_deg:
            pltpu.sync_copy(deg_sh.at[pl.ds(sid * RT, RT)],
                            degp_out.at[cid, pl.ds(sid * RT, RT)])

    if with_deg:
        out_type = (jax.ShapeDtypeStruct((NC, NPAD, D), jnp.float32),
                    jax.ShapeDtypeStruct((NC, NPAD), jnp.float32))
    else:
        out_type = jax.ShapeDtypeStruct((NC, NPAD, D), jnp.float32)
    scratch = [pltpu.VMEM_SHARED((NPAD, D), jnp.float32)]
    if with_deg:
        scratch.append(pltpu.VMEM_SHARED((NPAD,), jnp.float32))
    scratch += [
        pltpu.VMEM((2, 2, DG, K), jnp.int32),
        pltpu.VMEM((NBUF, K, D), jnp.float32),
    ]
    if with_deg:
        scratch.append(pltpu.VMEM((K,), jnp.float32))
    scratch += [pltpu.SemaphoreType.DMA((NBUF,)),
                pltpu.SemaphoreType.DMA((NBUF,))]
    if with_deg:
        scratch.append(pltpu.SemaphoreType.DMA((NBUF,)))
    scratch.append(pltpu.SemaphoreType.DMA((2,)))
    scratch.append(pltpu.SemaphoreType.DMA)
    return pl.kernel(
        body,
        out_type=out_type,
        mesh=plsc.VectorSubcoreMesh(core_axis_name="c", subcore_axis_name="s"),
        scratch_types=tuple(scratch),
    )


_sc_agg1 = _make_agg(True)
_sc_agg2 = _make_agg(False)


def _combine_body(p_ref, degp_ref, x1_ref):
    i = pl.program_id(0)
    deg = degp_ref[0, pl.ds(i * RB, RB)] + degp_ref[1, pl.ds(i * RB, RB)]
    rec = 1.0 / jnp.maximum(deg, 1.0)
    x1_ref[...] = (p_ref[0] + p_ref[1]) * rec[:, None]


def _tc_combine(p, degp):
    return pl.pallas_call(
        _combine_body,
        grid=(NBLK,),
        in_specs=[
            pl.BlockSpec((NC, RB, D), lambda i: (0, i, 0)),
            pl.BlockSpec((NC, NPAD), lambda i: (0, 0)),
        ],
        out_specs=pl.BlockSpec((RB, D), lambda i: (i, 0)),
        out_shape=jax.ShapeDtypeStruct((NPAD, D), jnp.float32),
    )(p, degp)


def _final_body(p_ref, degp_ref, x1_ref, out_ref):
    i = pl.program_id(0)
    deg = degp_ref[0, pl.ds(i * RBO, RBO)] + degp_ref[1, pl.ds(i * RBO, RBO)]
    rec = 1.0 / jnp.maximum(deg, 1.0)
    x2 = (p_ref[0] + p_ref[1]) * rec[:, None]
    x1 = x1_ref[...]
    dot = jnp.sum(x1 * x2, axis=1, keepdims=True)
    n1 = jnp.sqrt(jnp.sum(x1 * x1, axis=1, keepdims=True))
    n2 = jnp.sqrt(jnp.sum(x2 * x2, axis=1, keepdims=True))
    w = dot / (jnp.maximum(n1, 1e-8) * jnp.maximum(n2, 1e-8))
    out_ref[...] = w * x2 + (1.0 - w) * x1


def _tc_final(p2, degp, x1):
    return pl.pallas_call(
        _final_body,
        grid=(NBLKO,),
        in_specs=[
            pl.BlockSpec((NC, RBO, D), lambda i: (0, i, 0)),
            pl.BlockSpec((NC, NPAD), lambda i: (0, 0)),
            pl.BlockSpec((RBO, D), lambda i: (i, 0)),
        ],
        out_specs=pl.BlockSpec((RBO, D), lambda i: (i, 0)),
        out_shape=jax.ShapeDtypeStruct((N, D), jnp.float32),
    )(p2, degp, x1)


def kernel(features, edge_index):
    edge5 = edge_index.reshape(2, NW, DB, DG, K)

    p1, degp = _sc_agg1(features, edge5)
    x1 = _tc_combine(p1, degp)
    p2 = _sc_agg2(x1, edge5)
    return _tc_final(p2, degp, x1)


# speedup vs baseline: 1.0772x; 1.0772x over previous
import functools

import jax
import jax.numpy as jnp
from jax import lax
from jax.experimental import pallas as pl
from jax.experimental.pallas import tpu as pltpu
from jax.experimental.pallas import tpu_sc as plsc

N = 10000
E = 320000
D = 128

NC = 2
NS = 16
NW = NC * NS

NPAD = 10240
RT = NPAD // NS
K = 80
EW = E // NW
CW = EW // K
NBUF = 4
GLEAD = 2
SLAG = 2
DG = 5
DB = CW // DG

RB = 2048
NBLK = NPAD // RB
RBO = RB
NBLKO = NBLK


def _make_agg(with_deg):
    def body(*args):
        if with_deg:
            (x_hbm, edge_hbm,
             p_out, degp_out,
             acc_sh, deg_sh, ibuf, rows_v, ones_v,
             gsem, ssem, dsem, isem, zsem) = args
        else:
            (x_hbm, edge_hbm,
             p_out,
             acc_sh, ibuf, rows_v,
             gsem, ssem, isem, zsem) = args
        cid = lax.axis_index("c")
        sid = lax.axis_index("s")
        wid = sid * NC + cid

        pltpu.sync_copy(edge_hbm.at[0, wid, 0], ibuf.at[0, 0])
        pltpu.sync_copy(edge_hbm.at[1, wid, 0], ibuf.at[0, 1])

        def gather_idx(cn):
            return ibuf.at[lax.rem(lax.div(cn, DG), 2), 0, lax.rem(cn, DG)]

        for cn in range(GLEAD):
            pltpu.async_copy(x_hbm.at[gather_idx(cn)], rows_v.at[cn],
                             gsem.at[cn])

        z16 = jnp.zeros((16,), jnp.float32)

        def zfill(i, carry):
            for j in range(D // 16):
                rows_v[NBUF - 1, i, pl.ds(j * 16, 16)] = z16
            return carry

        lax.fori_loop(0, K, zfill, 0)
        if with_deg:
            one16 = jnp.full((16,), 1.0, jnp.float32)
            for j in range(K // 16):
                ones_v[pl.ds(j * 16, 16)] = one16
        zrow = rows_v.at[NBUF - 1]
        for t in range(RT // K):
            pltpu.async_copy(
                zrow, acc_sh.at[pl.ds(sid * RT + t * K, K)], zsem)
        if with_deg:
            for t in range(RT // D):
                pltpu.async_copy(
                    rows_v.at[NBUF - 1, 0],
                    deg_sh.at[pl.ds(sid * RT + t * D, D)], zsem)
        for t in range(RT // K):
            pltpu.make_async_copy(
                zrow, acc_sh.at[pl.ds(sid * RT, K)], zsem).wait()
        if with_deg:
            for t in range(RT // D):
                pltpu.make_async_copy(
                    rows_v.at[NBUF - 1, 0],
                    deg_sh.at[pl.ds(sid * RT, D)], zsem).wait()
        plsc.subcore_barrier()

        def chunk(ci, carry):
            b = lax.rem(ci, NBUF)
            g = lax.div(ci, DG)
            r = lax.rem(ci, DG)
            gb = lax.rem(g, 2)

            nb = lax.rem(ci + GLEAD, NBUF)

            @pl.when(ci >= SLAG)
            def _():
                pltpu.make_async_copy(
                    rows_v.at[0], acc_sh.at[ibuf.at[0, 1, 0]],
                    ssem.at[nb]).wait()
                if with_deg:
                    pltpu.make_async_copy(
                        ones_v, deg_sh.at[ibuf.at[0, 1, 0]],
                        dsem.at[nb]).wait()

            @pl.when(jnp.logical_and(r == 2, g + 1 < DB))
            def _():
                pltpu.async_copy(edge_hbm.at[0, wid, g + 1],
                                 ibuf.at[1 - gb, 0], isem.at[0])
                pltpu.async_copy(edge_hbm.at[1, wid, g + 1],
                                 ibuf.at[1 - gb, 1], isem.at[1])

            @pl.when(jnp.logical_and(r == 3, g + 1 < DB))
            def _():
                pltpu.make_async_copy(edge_hbm.at[0, wid, 0],
                                      ibuf.at[0, 0], isem.at[0]).wait()
                pltpu.make_async_copy(edge_hbm.at[1, wid, 0],
                                      ibuf.at[0, 1], isem.at[1]).wait()

            @pl.when(ci + GLEAD < CW)
            def _():
                pltpu.async_copy(x_hbm.at[gather_idx(ci + GLEAD)],
                                 rows_v.at[nb], gsem.at[nb])

            pltpu.make_async_copy(
                x_hbm.at[gather_idx(ci)], rows_v.at[b], gsem.at[b]).wait()
            didx = ibuf.at[gb, 1, r]
            pltpu.async_copy(rows_v.at[b], acc_sh.at[didx], ssem.at[b],
                             add=True)
            if with_deg:
                pltpu.async_copy(ones_v, deg_sh.at[didx], dsem.at[b],
                                 add=True)
            return carry

        lax.fori_loop(0, CW, chunk, 0)
        for k in range(SLAG):
            slot = (CW - SLAG + k) % NBUF
            pltpu.make_async_copy(
                rows_v.at[0], acc_sh.at[ibuf.at[0, 1, 0]],
                ssem.at[slot]).wait()
            if with_deg:
                pltpu.make_async_copy(
                    ones_v, deg_sh.at[ibuf.at[0, 1, 0]],
                    dsem.at[slot]).wait()
        plsc.subcore_barrier()

        pltpu.sync_copy(acc_sh.at[pl.ds(sid * RT, RT)],
                        p_out.at[cid, pl.ds(sid * RT, RT)])
        if with_deg:
            pltpu.sync_copy(deg_sh.at[pl.ds(sid * RT, RT)],
                            degp_out.at[cid, pl.ds(sid * RT, RT)])

    if with_deg:
        out_type = (jax.ShapeDtypeStruct((NC, NPAD, D), jnp.float32),
                    jax.ShapeDtypeStruct((NC, NPAD), jnp.float32))
    else:
        out_type = jax.ShapeDtypeStruct((NC, NPAD, D), jnp.float32)
    scratch = [pltpu.VMEM_SHARED((NPAD, D), jnp.float32)]
    if with_deg:
        scratch.append(pltpu.VMEM_SHARED((NPAD,), jnp.float32))
    scratch += [
        pltpu.VMEM((2, 2, DG, K), jnp.int32),
        pltpu.VMEM((NBUF, K, D), jnp.float32),
    ]
    if with_deg:
        scratch.append(pltpu.VMEM((K,), jnp.float32))
    scratch += [pltpu.SemaphoreType.DMA((NBUF,)),
                pltpu.SemaphoreType.DMA((NBUF,))]
    if with_deg:
        scratch.append(pltpu.SemaphoreType.DMA((NBUF,)))
    scratch.append(pltpu.SemaphoreType.DMA((2,)))
    scratch.append(pltpu.SemaphoreType.DMA)
    return pl.kernel(
        body,
        out_type=out_type,
        mesh=plsc.VectorSubcoreMesh(core_axis_name="c", subcore_axis_name="s"),
        scratch_types=tuple(scratch),
    )


_sc_agg1 = _make_agg(True)
_sc_agg2 = _make_agg(False)


def _combine_body(p_ref, degp_ref, x1_ref):
    i = pl.program_id(0)
    deg = degp_ref[0, pl.ds(i * RB, RB)] + degp_ref[1, pl.ds(i * RB, RB)]
    rec = 1.0 / jnp.maximum(deg, 1.0)
    x1_ref[...] = (p_ref[0] + p_ref[1]) * rec[:, None]


def _tc_combine(p, degp):
    return pl.pallas_call(
        _combine_body,
        grid=(NBLK,),
        in_specs=[
            pl.BlockSpec((NC, RB, D), lambda i: (0, i, 0)),
            pl.BlockSpec((NC, NPAD), lambda i: (0, 0)),
        ],
        out_specs=pl.BlockSpec((RB, D), lambda i: (i, 0)),
        out_shape=jax.ShapeDtypeStruct((NPAD, D), jnp.float32),
    )(p, degp)


def _final_body(p_ref, degp_ref, x1_ref, out_ref):
    i = pl.program_id(0)
    deg = degp_ref[0, pl.ds(i * RBO, RBO)] + degp_ref[1, pl.ds(i * RBO, RBO)]
    rec = 1.0 / jnp.maximum(deg, 1.0)
    x2 = (p_ref[0] + p_ref[1]) * rec[:, None]
    x1 = x1_ref[...]
    dot = jnp.sum(x1 * x2, axis=1, keepdims=True)
    n1 = jnp.sqrt(jnp.sum(x1 * x1, axis=1, keepdims=True))
    n2 = jnp.sqrt(jnp.sum(x2 * x2, axis=1, keepdims=True))
    w = dot / (jnp.maximum(n1, 1e-8) * jnp.maximum(n2, 1e-8))
    out_ref[...] = w * x2 + (1.0 - w) * x1


def _tc_final(p2, degp, x1):
    return pl.pallas_call(
        _final_body,
        grid=(NBLKO,),
        in_specs=[
            pl.BlockSpec((NC, RBO, D), lambda i: (0, i, 0)),
            pl.BlockSpec((NC, NPAD), lambda i: (0, 0)),
            pl.BlockSpec((RBO, D), lambda i: (i, 0)),
        ],
        out_specs=pl.BlockSpec((RBO, D), lambda i: (i, 0)),
        out_shape=jax.ShapeDtypeStruct((N, D), jnp.float32),
    )(p2, degp, x1)


def kernel(features, edge_index):
    edge5 = edge_index.reshape(2, NW, DB, DG, K)

    p1, degp = _sc_agg1(features, edge5)
    x1 = _tc_combine(p1, degp)
    p2 = _sc_agg2(x1, edge5)
    return _tc_final(p2, degp, x1)
